# baseline (speedup 1.0000x reference)
import jax
import jax.numpy as jnp
from jax import lax
from jax.experimental import pallas as pl
from jax.experimental.pallas import tpu as pltpu
from jax.experimental.pallas import tpu_sc as plsc

L = 16
NC = 2
NS = 16
NW = NC * NS
NCOL = 64
NROW_X = 1000000
NROWS_OUT = 16384
TOTAL = NROWS_OUT * NCOL
E = TOTAL // NW
CPW = NCOL // NW
GROUP = 128
NG = E // GROUP


def _body(x_hbm, idx_hbm, out_hbm, fidx_v, out_v, sem):
    wid = lax.axis_index("s") * NC + lax.axis_index("c")
    base = wid * E
    pltpu.sync_copy(idx_hbm.at[pl.ds(base, E)], fidx_v)

    def compute(col, carry):
        off = jnp.full((L,), (wid * CPW + col) * NROW_X, jnp.int32)
        run0 = col * NROWS_OUT

        def add_chunk(g, carry):
            p = run0 + g * L
            fidx_v[pl.ds(p, L)] = fidx_v[pl.ds(p, L)] + off
            return carry

        return lax.fori_loop(0, NROWS_OUT // L, add_chunk, carry)

    lax.fori_loop(0, CPW, compute, 0)

    def fire(r, carry):
        pltpu.async_copy(
            x_hbm.at[fidx_v.at[pl.ds(r * GROUP, GROUP)]],
            out_v.at[pl.ds(r * GROUP, GROUP)],
            sem,
        )
        return carry

    lax.fori_loop(0, NG, fire, 0)

    pltpu.make_async_copy(x_hbm.at[pl.ds(0, E)], out_v, sem).wait()

    pltpu.sync_copy(out_v, out_hbm.at[pl.ds(base, E)])


def kernel(x, dim, index):
    del dim
    xf = x.T.reshape(-1)
    idxf = index.astype(jnp.int32).T.reshape(-1)
    outf = pl.kernel(
        _body,
        out_type=jax.ShapeDtypeStruct((TOTAL,), jnp.float32),
        mesh=plsc.VectorSubcoreMesh(core_axis_name="c", subcore_axis_name="s"),
        compiler_params=pltpu.CompilerParams(needs_layout_passes=False),
        scratch_types=[
            pltpu.VMEM((E,), jnp.int32),
            pltpu.VMEM((E,), jnp.float32),
            pltpu.SemaphoreType.DMA,
        ],
    )(xf, idxf)
    return outf.reshape(NCOL, NROWS_OUT).T


# speedup vs baseline: 7.6321x; 7.6321x over previous
import jax
import jax.numpy as jnp
from jax import lax
from jax.experimental import pallas as pl
from jax.experimental.pallas import tpu as pltpu
from jax.experimental.pallas import tpu_sc as plsc

L = 16
NC = 2
NS = 16
NW = NC * NS
NCOL = 64
NROW_X = 1000000
NROWS_OUT = 16384
TOTAL = NROWS_OUT * NCOL
E = TOTAL // NW
CPW = NCOL // NW
GROUP = 128
NG = E // GROUP


def _body(x_hbm, idx_hbm, out_hbm, fidx_v, out_v, sem):
    wid = lax.axis_index("s") * NC + lax.axis_index("c")
    base = wid * E
    pltpu.sync_copy(idx_hbm.at[pl.ds(base, E)], fidx_v)

    def compute(col, carry):
        off = jnp.full((L,), wid * CPW + col, jnp.int32)
        run0 = col * NROWS_OUT

        def add_chunk(g, carry):
            p = run0 + g * L
            fidx_v[pl.ds(p, L)] = fidx_v[pl.ds(p, L)] * NCOL + off
            return carry

        return lax.fori_loop(0, NROWS_OUT // L, add_chunk, carry)

    lax.fori_loop(0, CPW, compute, 0)

    def fire(r, carry):
        pltpu.async_copy(
            x_hbm.at[fidx_v.at[pl.ds(r * GROUP, GROUP)]],
            out_v.at[pl.ds(r * GROUP, GROUP)],
            sem,
        )
        return carry

    lax.fori_loop(0, NG, fire, 0)

    pltpu.make_async_copy(x_hbm.at[pl.ds(0, E)], out_v, sem).wait()

    pltpu.sync_copy(out_v, out_hbm.at[pl.ds(base, E)])


def kernel(x, dim, index):
    del dim
    xf = x.reshape(-1)
    idxf = index.astype(jnp.int32).T.reshape(-1)
    outf = pl.kernel(
        _body,
        out_type=jax.ShapeDtypeStruct((TOTAL,), jnp.float32),
        mesh=plsc.VectorSubcoreMesh(core_axis_name="c", subcore_axis_name="s"),
        compiler_params=pltpu.CompilerParams(needs_layout_passes=False),
        scratch_types=[
            pltpu.VMEM((E,), jnp.int32),
            pltpu.VMEM((E,), jnp.float32),
            pltpu.SemaphoreType.DMA,
        ],
    )(xf, idxf)
    return outf.reshape(NCOL, NROWS_OUT).T
